# Initial kernel scaffold; baseline (speedup 1.0000x reference)
import functools

import jax
import jax.numpy as jnp
from jax import lax
from jax.experimental import pallas as pl
from jax.experimental.pallas import tpu as pltpu
from jax.experimental.pallas import tpu_sc as plsc

B, F, V, K = 16384, 26, 1000000, 10
NC, NS = 2, 16
NW = NC * NS
SPW = B // NW
CH = 128
NSUB = SPW // CH
CF = CH * F
NG = CH // 16


def _fm_body(idx_hbm, vals_hbm, w_hbm, v_hbm, b_hbm, out_hbm,
             idx_v, vals_v, vrows, wrows, b_v, out_v, sem_v, sem_w):
    c = lax.axis_index("c")
    s = lax.axis_index("s")
    wid = s * NC + c
    base_s = wid * SPW

    pltpu.sync_copy(b_hbm, b_v)
    bvec = b_v[...]
    iota = lax.iota(jnp.int32, 16)
    kcols = [jnp.full((16,), k, dtype=jnp.int32) for k in range(K)]
    zcol = jnp.zeros((16,), dtype=jnp.int32)
    zf = jnp.zeros((16,), dtype=jnp.float32)

    for sub in range(NSUB):
        off = (base_s + sub * CH) * F
        pltpu.sync_copy(idx_hbm.at[pl.ds(off, CF)], idx_v)
        pltpu.sync_copy(vals_hbm.at[pl.ds(off, CF)], vals_v)
        cp_v = pltpu.async_copy(v_hbm.at[idx_v], vrows, sem_v)
        cp_w = pltpu.async_copy(w_hbm.at[idx_v], wrows, sem_w)
        cp_v.wait()
        cp_w.wait()

        def group(g, _):
            rbase = (g * 16 + iota) * F

            def fstep(f, carry):
                accw = carry[0]
                acc = list(carry[1:1 + K])
                acc2 = list(carry[1 + K:])
                r = rbase + f
                vf = plsc.load_gather(vals_v, [r])
                wv = plsc.load_gather(wrows, [r, zcol])
                accw = accw + vf * wv
                for k in range(K):
                    x = plsc.load_gather(vrows, [r, kcols[k]])
                    t = vf * x
                    acc[k] = acc[k] + t
                    acc2[k] = acc2[k] + t * t
                return (accw,) + tuple(acc) + tuple(acc2)

            carry = lax.fori_loop(0, F, fstep, (zf,) * (1 + 2 * K))
            accw = carry[0]
            p = zf
            for k in range(K):
                p = p + (carry[1 + k] * carry[1 + k] - carry[1 + K + k])
            logit = accw + bvec + 0.5 * p
            y = 1.0 / (1.0 + jnp.exp(-logit))
            out_v[pl.ds(g * 16, 16)] = y
            return 0

        lax.fori_loop(0, NG, group, 0)
        pltpu.sync_copy(out_v, out_hbm.at[pl.ds(base_s + sub * CH, CH)])


@functools.partial(
    pl.kernel,
    out_type=jax.ShapeDtypeStruct((B,), jnp.float32),
    mesh=plsc.VectorSubcoreMesh(core_axis_name="c", subcore_axis_name="s"),
    scratch_types=[
        pltpu.VMEM((CF,), jnp.int32),
        pltpu.VMEM((CF,), jnp.float32),
        pltpu.VMEM((CF, K), jnp.float32),
        pltpu.VMEM((CF, 1), jnp.float32),
        pltpu.VMEM((16,), jnp.float32),
        pltpu.VMEM((CH,), jnp.float32),
        pltpu.SemaphoreType.DMA,
        pltpu.SemaphoreType.DMA,
    ],
)
def _fm_kernel(idx_hbm, vals_hbm, w_hbm, v_hbm, b_hbm, out_hbm, *rest):
    _fm_body(idx_hbm, vals_hbm, w_hbm, v_hbm, b_hbm, out_hbm, *rest)


def kernel(indices, values, w, v, b):
    idx_flat = indices.reshape(-1).astype(jnp.int32)
    vals_flat = values.reshape(-1).astype(jnp.float32)
    b16 = jnp.broadcast_to(b.astype(jnp.float32).reshape(1), (16,))
    return _fm_kernel(idx_flat, vals_flat, w, v, b16)


# speedup vs baseline: 1.1530x; 1.1530x over previous
import functools

import jax
import jax.numpy as jnp
from jax import lax
from jax.experimental import pallas as pl
from jax.experimental.pallas import tpu as pltpu
from jax.experimental.pallas import tpu_sc as plsc

B, F, V, K = 16384, 26, 1000000, 10
NC, NS = 2, 16
NW = NC * NS
SPW = B // NW
CH = 32
NCH = SPW // CH
CF = CH * F
NG = CH // 16
GRAN_V = V * K // 16
GRAN_W = V // 16


def _fm_body(idx_hbm, vals_hbm, w_hbm, v_hbm, b_hbm, out_hbm,
             idx_v, vals_v, ig_v, gw_v, buf, bufw, b_v, out_v, sems):
    c = lax.axis_index("c")
    s = lax.axis_index("s")
    wid = s * NC + c
    base = wid * SPW * F

    pltpu.sync_copy(b_hbm, b_v)
    bvec = b_v[...]
    iota = lax.iota(jnp.int32, 16)

    def stage_in(slot, ch):
        off = base + ch * CF
        pltpu.sync_copy(idx_hbm.at[pl.ds(off, CF)], idx_v.at[slot])
        pltpu.sync_copy(vals_hbm.at[pl.ds(off, CF)], vals_v.at[slot])

        def build(t, _):
            ii = idx_v[slot, pl.ds(t * 16, 16)]
            woff = ii * 10
            g0 = lax.shift_right_logical(woff, 4)
            g1 = jnp.minimum(g0 + 1, GRAN_V - 1)
            pos = t * 32 + 2 * iota
            plsc.store_scatter(ig_v.at[slot], [pos], g0)
            plsc.store_scatter(ig_v.at[slot], [pos + 1], g1)
            gw_v[slot, pl.ds(t * 16, 16)] = lax.shift_right_logical(ii, 4)
            return 0

        lax.fori_loop(0, CF // 16, build, 0)
        pltpu.async_copy(v_hbm.at[ig_v.at[slot]], buf.at[slot], sems.at[slot, 0])
        pltpu.async_copy(w_hbm.at[gw_v.at[slot]], bufw.at[slot], sems.at[slot, 1])

    def wait_in(slot):
        pltpu.make_async_copy(
            v_hbm.at[ig_v.at[slot]], buf.at[slot], sems.at[slot, 0]).wait()
        pltpu.make_async_copy(
            w_hbm.at[gw_v.at[slot]], bufw.at[slot], sems.at[slot, 1]).wait()

    def compute(slot, ch):
        def group(g, _):
            rbase = (g * 16 + iota) * F

            def fstep(f, carry):
                accw = carry[0]
                acc = list(carry[1:1 + K])
                acc2 = list(carry[1 + K:])
                r = rbase + f
                ii = plsc.load_gather(idx_v.at[slot], [r])
                vf = plsc.load_gather(vals_v.at[slot], [r])
                ow = jnp.bitwise_and(ii, 15)
                o = jnp.bitwise_and(ii * 10, 15)
                wv = plsc.load_gather(bufw.at[slot], [r, ow])
                accw = accw + vf * wv
                r2 = 2 * r
                for k in range(K):
                    x = plsc.load_gather(buf.at[slot], [r2, o + k])
                    t = vf * x
                    acc[k] = acc[k] + t
                    acc2[k] = acc2[k] + t * t
                return (accw,) + tuple(acc) + tuple(acc2)

            zf = jnp.zeros((16,), jnp.float32)
            carry = lax.fori_loop(0, F, fstep, (zf,) * (1 + 2 * K))
            accw = carry[0]
            p = zf
            for k in range(K):
                p = p + (carry[1 + k] * carry[1 + k] - carry[1 + K + k])
            logit = accw + bvec + 0.5 * p
            y = 1.0 / (1.0 + jnp.exp(-logit))
            out_v[pl.ds(ch * CH + g * 16, 16)] = y
            return 0

        lax.fori_loop(0, NG, group, 0)

    stage_in(0, 0)

    def superstep(t, _):
        c0 = 2 * t
        stage_in(1, c0 + 1)
        wait_in(0)
        compute(0, c0)

        @pl.when(t < NCH // 2 - 1)
        def _():
            stage_in(0, c0 + 2)

        wait_in(1)
        compute(1, c0 + 1)
        return 0

    lax.fori_loop(0, NCH // 2, superstep, 0)
    pltpu.sync_copy(out_v, out_hbm.at[pl.ds(wid * SPW, SPW)])


@functools.partial(
    pl.kernel,
    out_type=jax.ShapeDtypeStruct((B,), jnp.float32),
    mesh=plsc.VectorSubcoreMesh(core_axis_name="c", subcore_axis_name="s"),
    scratch_types=[
        pltpu.VMEM((2, CF), jnp.int32),
        pltpu.VMEM((2, CF), jnp.float32),
        pltpu.VMEM((2, 2 * CF), jnp.int32),
        pltpu.VMEM((2, CF), jnp.int32),
        pltpu.VMEM((2, 2 * CF, 16), jnp.float32),
        pltpu.VMEM((2, CF, 16), jnp.float32),
        pltpu.VMEM((16,), jnp.float32),
        pltpu.VMEM((SPW,), jnp.float32),
        pltpu.SemaphoreType.DMA((2, 2)),
    ],
    compiler_params=pltpu.CompilerParams(
        needs_layout_passes=False, use_tc_tiling_on_sc=False),
)
def _fm_kernel(idx_hbm, vals_hbm, w_hbm, v_hbm, b_hbm, out_hbm, *rest):
    _fm_body(idx_hbm, vals_hbm, w_hbm, v_hbm, b_hbm, out_hbm, *rest)


def kernel(indices, values, w, v, b):
    idx_flat = indices.reshape(-1).astype(jnp.int32)
    vals_flat = values.reshape(-1).astype(jnp.float32)
    w16 = w.reshape(GRAN_W, 16)
    v16 = v.reshape(GRAN_V, 16)
    b16 = jnp.broadcast_to(b.astype(jnp.float32).reshape(1), (16,))
    return _fm_kernel(idx_flat, vals_flat, w16, v16, b16)


# speedup vs baseline: 1.5823x; 1.3723x over previous
import functools

import jax
import jax.numpy as jnp
from jax import lax
from jax.experimental import pallas as pl
from jax.experimental.pallas import tpu as pltpu
from jax.experimental.pallas import tpu_sc as plsc

B, F, V, K = 16384, 26, 1000000, 10
NC, NS = 2, 16
NW = NC * NS
SPW = B // NW
CH = 32
NCH = SPW // CH
CF = CH * F
NG = CH // 16
GRAN_W = V // 16

PCH = 2000
NPCH = V // PCH
PPT = (NPCH + NW - 1) // NW


def _pack_body(*refs):
    cols = refs[:K]
    pt_hbm = refs[K]
    colbuf, rowbuf, sem, osem = refs[K + 1:]
    c = lax.axis_index("c")
    s = lax.axis_index("s")
    wid = s * NC + c
    iota = lax.iota(jnp.int32, 16)

    def chunk(t, _):
        cid = t * NW + wid

        @pl.when(cid < NPCH)
        def _():
            base = cid * PCH
            for k in range(K):
                pltpu.async_copy(cols[k].at[pl.ds(base, PCH)],
                                 colbuf.at[k], sem)
            for k in range(K):
                pltpu.make_async_copy(cols[k].at[pl.ds(base, PCH)],
                                      colbuf.at[k], sem).wait()

            def fill(u, _):
                pos = (u * 16 + iota) * 16
                for k in range(K):
                    xc = colbuf[k, pl.ds(u * 16, 16)]
                    plsc.store_scatter(rowbuf, [pos + k], xc)
                return 0

            lax.fori_loop(0, PCH // 16, fill, 0)
            pltpu.async_copy(rowbuf, pt_hbm.at[pl.ds(base * 16, PCH * 16)],
                             osem)
            pltpu.make_async_copy(rowbuf,
                                  pt_hbm.at[pl.ds(base * 16, PCH * 16)],
                                  osem).wait()
        return 0

    lax.fori_loop(0, PPT, chunk, 0)


_scpack = functools.partial(
    pl.kernel,
    out_type=jax.ShapeDtypeStruct((16 * V,), jnp.float32),
    mesh=plsc.VectorSubcoreMesh(core_axis_name="c", subcore_axis_name="s"),
    scratch_types=[
        pltpu.VMEM((K, PCH), jnp.float32),
        pltpu.VMEM((16 * PCH,), jnp.float32),
        pltpu.SemaphoreType.DMA,
        pltpu.SemaphoreType.DMA,
    ],
    compiler_params=pltpu.CompilerParams(
        needs_layout_passes=False, use_tc_tiling_on_sc=False),
)(_pack_body)


def _fm_body(idx_hbm, vals_hbm, w_hbm, v_hbm, b_hbm, out_hbm,
             idx_v, vals_v, gw_v, buf, bufw, b_v, out_v, sems):
    c = lax.axis_index("c")
    s = lax.axis_index("s")
    wid = s * NC + c

    pltpu.sync_copy(b_hbm, b_v)
    bvec = b_v[...]
    iota = lax.iota(jnp.int32, 16)
    kcols = [jnp.full((16,), k, dtype=jnp.int32) for k in range(K)]

    def stage_in(slot, ch):
        row = wid * NCH + ch
        pltpu.sync_copy(idx_hbm.at[row], idx_v.at[slot])
        pltpu.sync_copy(vals_hbm.at[row], vals_v.at[slot])

        def build(t, _):
            ii = idx_v[slot, pl.ds(t * 16, 16)]
            gw_v[slot, pl.ds(t * 16, 16)] = lax.shift_right_logical(ii, 4)
            return 0

        lax.fori_loop(0, CF // 16, build, 0)
        pltpu.async_copy(v_hbm.at[idx_v.at[slot]], buf.at[slot],
                         sems.at[slot, 0])
        pltpu.async_copy(w_hbm.at[gw_v.at[slot]], bufw.at[slot],
                         sems.at[slot, 1])

    def wait_in(slot):
        pltpu.make_async_copy(
            v_hbm.at[idx_v.at[slot]], buf.at[slot], sems.at[slot, 0]).wait()
        pltpu.make_async_copy(
            w_hbm.at[gw_v.at[slot]], bufw.at[slot], sems.at[slot, 1]).wait()

    def compute(slot, ch):
        def group(g, _):
            rbase = (g * 16 + iota) * F

            def fstep(f, carry):
                accw = carry[0]
                acc = list(carry[1:1 + K])
                acc2 = list(carry[1 + K:])
                r = rbase + f
                ii = plsc.load_gather(idx_v.at[slot], [r])
                vf = plsc.load_gather(vals_v.at[slot], [r])
                ow = jnp.bitwise_and(ii, 15)
                wv = plsc.load_gather(bufw.at[slot], [r, ow])
                accw = accw + vf * wv
                for k in range(K):
                    x = plsc.load_gather(buf.at[slot], [r, kcols[k]])
                    t = vf * x
                    acc[k] = acc[k] + t
                    acc2[k] = acc2[k] + t * t
                return (accw,) + tuple(acc) + tuple(acc2)

            zf = jnp.zeros((16,), jnp.float32)
            carry = lax.fori_loop(0, F, fstep, (zf,) * (1 + 2 * K))
            accw = carry[0]
            p = zf
            for k in range(K):
                p = p + (carry[1 + k] * carry[1 + k] - carry[1 + K + k])
            logit = accw + bvec + 0.5 * p
            y = 1.0 / (1.0 + jnp.exp(-logit))
            out_v[pl.ds(ch * CH + g * 16, 16)] = y
            return 0

        lax.fori_loop(0, NG, group, 0)

    stage_in(0, 0)

    def superstep(t, _):
        c0 = 2 * t
        stage_in(1, c0 + 1)
        wait_in(0)
        compute(0, c0)

        @pl.when(t < NCH // 2 - 1)
        def _():
            stage_in(0, c0 + 2)

        wait_in(1)
        compute(1, c0 + 1)
        return 0

    lax.fori_loop(0, NCH // 2, superstep, 0)
    pltpu.sync_copy(out_v, out_hbm.at[pl.ds(wid * SPW, SPW)])


@functools.partial(
    pl.kernel,
    out_type=jax.ShapeDtypeStruct((B,), jnp.float32),
    mesh=plsc.VectorSubcoreMesh(core_axis_name="c", subcore_axis_name="s"),
    scratch_types=[
        pltpu.VMEM((2, CF), jnp.int32),
        pltpu.VMEM((2, CF), jnp.float32),
        pltpu.VMEM((2, CF), jnp.int32),
        pltpu.VMEM((2, CF, 16), jnp.float32),
        pltpu.VMEM((2, CF, 16), jnp.float32),
        pltpu.VMEM((16,), jnp.float32),
        pltpu.VMEM((SPW,), jnp.float32),
        pltpu.SemaphoreType.DMA((2, 2)),
    ],
    compiler_params=pltpu.CompilerParams(
        needs_layout_passes=False, use_tc_tiling_on_sc=False),
)
def _fm_kernel(idx_hbm, vals_hbm, w_hbm, v_hbm, b_hbm, out_hbm, *rest):
    _fm_body(idx_hbm, vals_hbm, w_hbm, v_hbm, b_hbm, out_hbm, *rest)


def kernel(indices, values, w, v, b):
    idx2 = indices.reshape(B * F // CF, CF).astype(jnp.int32)
    vals2 = values.reshape(B * F // CF, CF).astype(jnp.float32)
    cols = [v[:, k] for k in range(K)]
    pt = _scpack(*cols).reshape(V, 16)
    w16 = w.reshape(GRAN_W, 16)
    b16 = jnp.broadcast_to(b.astype(jnp.float32).reshape(1), (16,))
    return _fm_kernel(idx2, vals2, w16, pt, b16)
